# baseline (speedup 1.0000x reference)
import functools

import jax
import jax.numpy as jnp
from jax import lax
from jax.experimental import pallas as pl
from jax.experimental.pallas import tpu as pltpu
from jax.experimental.pallas import tpu_sc as plsc

NUM_CODES = 1024
DIM = 64
ROWS = 1024


def _vq_block(z_ref, sw_ref, w_ref, idx_ref):
    zb = z_ref[...].reshape(ROWS, DIM)
    w = w_ref[...]
    sz = jnp.sum(zb * zb, axis=1, keepdims=True)
    sw = sw_ref[...]
    m2 = jax.lax.dot_general(
        (zb * -2.0).astype(jnp.bfloat16), w.astype(jnp.bfloat16),
        (((1,), (1,)), ((), ())),
        preferred_element_type=jnp.float32,
    )
    CH = 128
    nch = NUM_CODES // CH
    runval = runchunk = None
    for c in range(nch):
        dc = (sz + sw[:, c * CH:(c + 1) * CH]) + m2[:, c * CH:(c + 1) * CH]
        if c == 0:
            runval = dc
            runchunk = jnp.zeros((ROWS, CH), jnp.float32)
        else:
            upd = dc < runval
            runval = jnp.where(upd, dc, runval)
            runchunk = jnp.where(upd, jnp.float32(c), runchunk)
    gmin = jnp.min(runval, axis=1, keepdims=True)
    lane = jax.lax.broadcasted_iota(jnp.int32, (ROWS, CH), 1).astype(jnp.float32)
    cand = runchunk * CH + lane
    idx = jnp.min(jnp.where(runval == gmin, cand, float(NUM_CODES)),
                  axis=1, keepdims=True)
    idx_ref[...] = idx.astype(jnp.int32).reshape(idx_ref.shape)


def _tc_argmin(z, sw, W):
    nblk = (z.shape[0] * z.shape[1]) // ROWS
    zf = z.reshape(nblk, ROWS, DIM)
    idx3 = pl.pallas_call(
        _vq_block,
        grid=(nblk,),
        in_specs=[
            pl.BlockSpec((1, ROWS, DIM), lambda b: (b, 0, 0)),
            pl.BlockSpec((1, NUM_CODES), lambda b: (0, 0)),
            pl.BlockSpec((NUM_CODES, DIM), lambda b: (0, 0)),
        ],
        out_specs=pl.BlockSpec((1, ROWS, 1), lambda b: (b, 0, 0)),
        out_shape=jax.ShapeDtypeStruct((nblk, ROWS, 1), jnp.int32),
    )(zf, sw, W)
    return idx3.reshape(-1)


_SC_INFO = plsc.get_sparse_core_info()
_NC = _SC_INFO.num_cores
_NS = _SC_INFO.num_subcores
_NW = _NC * _NS


def _make_sc_gather(B):
    b_per_w = B // _NW
    mesh = plsc.VectorSubcoreMesh(core_axis_name="c", subcore_axis_name="s")

    @functools.partial(
        pl.kernel, mesh=mesh,
        compiler_params=pltpu.CompilerParams(use_tc_tiling_on_sc=False),
        out_type=jax.ShapeDtypeStruct((B, DIM), jnp.float32),
        scratch_types=[
            pltpu.VMEM((b_per_w,), jnp.int32),
            pltpu.VMEM((b_per_w, DIM), jnp.float32),
            pltpu.SemaphoreType.DMA,
        ],
    )
    def sc_gather(table_hbm, idx_hbm, out_hbm, idx_v, rows_v, sem):
        wid = lax.axis_index("s") * _NC + lax.axis_index("c")
        base = wid * b_per_w
        pltpu.sync_copy(idx_hbm.at[pl.ds(base, b_per_w)], idx_v)
        pltpu.async_copy(table_hbm.at[idx_v], rows_v, sem).wait()
        pltpu.sync_copy(rows_v, out_hbm.at[pl.ds(base, b_per_w)])

    return sc_gather


def kernel(z, W):
    B, HW, D = z.shape
    sw = jnp.sum(W ** 2, axis=1).reshape(1, NUM_CODES)
    half = B // 2
    gather = _make_sc_gather(half * HW)
    idx0 = _tc_argmin(z[:half], sw, W)
    zq0 = gather(W, idx0)
    idx1 = _tc_argmin(z[half:], sw, W)
    zq1 = gather(W, idx1)
    idx = jnp.concatenate([idx0, idx1])
    zq = jnp.concatenate([zq0, zq1])
    return zq.reshape(B, HW, D), idx.reshape(B, HW)


# speedup vs baseline: 1.1000x; 1.1000x over previous
import functools

import jax
import jax.numpy as jnp
from jax import lax
from jax.experimental import pallas as pl
from jax.experimental.pallas import tpu as pltpu
from jax.experimental.pallas import tpu_sc as plsc

NUM_CODES = 1024
DIM = 64
ROWS = 1024


def _vq_block(z_ref, sw_ref, w_ref, idx_ref):
    zb = z_ref[...].reshape(ROWS, DIM)
    w = w_ref[...]
    sz = jnp.sum(zb * zb, axis=1, keepdims=True)
    sw = sw_ref[...]
    m2 = jax.lax.dot_general(
        (zb * -2.0).astype(jnp.bfloat16), w.astype(jnp.bfloat16),
        (((1,), (1,)), ((), ())),
        preferred_element_type=jnp.float32,
    )
    CH = 128
    nch = NUM_CODES // CH
    runval = runchunk = None
    for c in range(nch):
        dc = (sz + sw[:, c * CH:(c + 1) * CH]) + m2[:, c * CH:(c + 1) * CH]
        if c == 0:
            runval = dc
            runchunk = jnp.zeros((ROWS, CH), jnp.float32)
        else:
            upd = dc < runval
            runval = jnp.where(upd, dc, runval)
            runchunk = jnp.where(upd, jnp.float32(c), runchunk)
    gmin = jnp.min(runval, axis=1, keepdims=True)
    lane = jax.lax.broadcasted_iota(jnp.int32, (ROWS, CH), 1).astype(jnp.float32)
    cand = runchunk * CH + lane
    idx = jnp.min(jnp.where(runval == gmin, cand, float(NUM_CODES)),
                  axis=1, keepdims=True)
    idx_ref[...] = idx.astype(jnp.int32).reshape(idx_ref.shape)


def _tc_argmin(z, sw, W):
    nblk = (z.shape[0] * z.shape[1]) // ROWS
    zf = z.reshape(nblk, ROWS, DIM)
    idx3 = pl.pallas_call(
        _vq_block,
        grid=(nblk,),
        in_specs=[
            pl.BlockSpec((1, ROWS, DIM), lambda b: (b, 0, 0)),
            pl.BlockSpec((1, NUM_CODES), lambda b: (0, 0)),
            pl.BlockSpec((NUM_CODES, DIM), lambda b: (0, 0)),
        ],
        out_specs=pl.BlockSpec((1, ROWS, 1), lambda b: (b, 0, 0)),
        out_shape=jax.ShapeDtypeStruct((nblk, ROWS, 1), jnp.int32),
    )(zf, sw, W)
    return idx3.reshape(-1)


_SC_INFO = plsc.get_sparse_core_info()
_NC = _SC_INFO.num_cores
_NS = _SC_INFO.num_subcores
_NW = _NC * _NS


def _make_sc_gather(B):
    b_per_w = B // _NW
    mesh = plsc.VectorSubcoreMesh(core_axis_name="c", subcore_axis_name="s")

    @functools.partial(
        pl.kernel, mesh=mesh,
        compiler_params=pltpu.CompilerParams(use_tc_tiling_on_sc=False),
        out_type=jax.ShapeDtypeStruct((B, DIM), jnp.float32),
        scratch_types=[
            pltpu.VMEM((b_per_w,), jnp.int32),
            pltpu.VMEM((b_per_w, DIM), jnp.float32),
            pltpu.SemaphoreType.DMA,
        ],
    )
    def sc_gather(table_hbm, idx_hbm, out_hbm, idx_v, rows_v, sem):
        wid = lax.axis_index("s") * _NC + lax.axis_index("c")
        base = wid * b_per_w
        pltpu.sync_copy(idx_hbm.at[pl.ds(base, b_per_w)], idx_v)
        pltpu.async_copy(table_hbm.at[idx_v], rows_v, sem).wait()
        pltpu.sync_copy(rows_v, out_hbm.at[pl.ds(base, b_per_w)])

    return sc_gather


def kernel(z, W):
    B, HW, D = z.shape
    sw = jnp.sum(W ** 2, axis=1).reshape(1, NUM_CODES)
    idx = _tc_argmin(z, sw, W)
    zq = _make_sc_gather(B * HW)(W, idx)
    return zq.reshape(B, HW, D), idx.reshape(B, HW)
